# baseline (speedup 1.0000x reference)
import functools

import jax
import jax.numpy as jnp
from jax import lax
from jax.experimental import pallas as pl
from jax.experimental.pallas import tpu as pltpu, tpu_sc as plsc

_D = 128
_C = 128
_GPB = 2
_RB = _C * _GPB
_NBUF = 3


@functools.lru_cache(maxsize=None)
def _make_gather(total):
    info = plsc.get_sparse_core_info()
    nc, ns = info.num_cores, info.num_subcores
    nw = nc * ns
    b_per_w = total // nw
    n_idx_rows = b_per_w // _C
    n_chunks = b_per_w // _RB
    assert n_chunks >= 2 * _NBUF
    mesh = plsc.VectorSubcoreMesh(core_axis_name="c", subcore_axis_name="s")

    @functools.partial(
        pl.kernel,
        mesh=mesh,
        out_type=jax.ShapeDtypeStruct((total, _D), jnp.float32),
        scratch_types=[
            pltpu.VMEM((n_idx_rows, _C), jnp.int32),
        ]
        + [pltpu.VMEM((_RB, _D), jnp.float32) for _ in range(_NBUF)]
        + [pltpu.SemaphoreType.DMA for _ in range(2 * _NBUF)],
    )
    def gather_kernel(idx_hbm, table_hbm, out_hbm, idx_v, *rest):
        bufs = rest[:_NBUF]
        gs = rest[_NBUF:2 * _NBUF]
        ss = rest[2 * _NBUF:]
        wid = lax.axis_index("s") * nc + lax.axis_index("c")
        base = wid * b_per_w
        pltpu.sync_copy(idx_hbm.at[wid], idx_v)

        def start_gather(i, b):
            for g in range(_GPB):
                pltpu.async_copy(table_hbm.at[idx_v.at[i * _GPB + g]],
                                 bufs[b].at[pl.ds(g * _C, _C)], gs[b])

        def wait_gather(b):
            pltpu.make_async_copy(
                table_hbm.at[pl.ds(0, _RB)], bufs[b], gs[b]).wait()

        def start_store(i, b):
            pltpu.async_copy(bufs[b], out_hbm.at[pl.ds(base + i * _RB, _RB)],
                             ss[b])

        def wait_store(b):
            pltpu.make_async_copy(bufs[b], out_hbm.at[pl.ds(base, _RB)],
                                  ss[b]).wait()

        def step(i, b, store_wait=True, lookahead=True):
            wait_gather(b)
            start_store(i, b)
            if lookahead:
                pb = (b - 1) % _NBUF
                if store_wait:
                    wait_store(pb)
                start_gather(i + _NBUF - 1, pb)

        for b in range(_NBUF - 1):
            start_gather(b, b)
        step(0, 0, store_wait=False)

        n_main = (n_chunks - 1 - _NBUF) // _NBUF

        def body(t, carry):
            for j in range(_NBUF):
                i = 1 + _NBUF * t + j
                step(i, (1 + j) % _NBUF)
            return carry

        lax.fori_loop(0, n_main, body, 0)

        for i in range(1 + _NBUF * n_main, n_chunks):
            step(i, i % _NBUF, lookahead=(i + _NBUF - 1 < n_chunks))
        for i in range(n_chunks - _NBUF, n_chunks):
            wait_store(i % _NBUF)

    return gather_kernel


@jax.jit
def kernel(x, embedding):
    b, s = x.shape
    total = b * s
    flat = x.reshape(32, total // (32 * _C), _C)
    out = _make_gather(total)(flat, embedding)
    return out.reshape(b, s, _D)


# speedup vs baseline: 1.0236x; 1.0236x over previous
import functools

import jax
import jax.numpy as jnp
from jax import lax
from jax.experimental import pallas as pl
from jax.experimental.pallas import tpu as pltpu, tpu_sc as plsc

_D = 128
_C = 128
_NBUF = 7
_LOOK = 5


@functools.lru_cache(maxsize=None)
def _make_gather(total):
    info = plsc.get_sparse_core_info()
    nc, ns = info.num_cores, info.num_subcores
    nw = nc * ns
    b_per_w = total // nw
    n_chunks = b_per_w // _C
    pro = _NBUF - _LOOK
    assert n_chunks >= 2 * _NBUF
    mesh = plsc.VectorSubcoreMesh(core_axis_name="c", subcore_axis_name="s")

    @functools.partial(
        pl.kernel,
        mesh=mesh,
        out_type=jax.ShapeDtypeStruct((total, _D), jnp.float32),
        scratch_types=[
            pltpu.VMEM((n_chunks, _C), jnp.int32),
        ]
        + [pltpu.VMEM((_C, _D), jnp.float32) for _ in range(_NBUF)]
        + [pltpu.SemaphoreType.DMA for _ in range(2 * _NBUF)],
    )
    def gather_kernel(idx_hbm, table_hbm, out_hbm, idx_v, *rest):
        bufs = rest[:_NBUF]
        gs = rest[_NBUF:2 * _NBUF]
        ss = rest[2 * _NBUF:]
        wid = lax.axis_index("s") * nc + lax.axis_index("c")
        base = wid * b_per_w
        pltpu.sync_copy(idx_hbm.at[wid], idx_v)

        def start_gather(i, b):
            pltpu.async_copy(table_hbm.at[idx_v.at[i]], bufs[b], gs[b])

        def wait_gather(b):
            pltpu.make_async_copy(
                table_hbm.at[pl.ds(0, _C)], bufs[b], gs[b]).wait()

        def start_store(i, b):
            pltpu.async_copy(bufs[b], out_hbm.at[pl.ds(base + i * _C, _C)],
                             ss[b])

        def wait_store(b):
            pltpu.make_async_copy(bufs[b], out_hbm.at[pl.ds(base, _C)],
                                  ss[b]).wait()

        def step(i, b, store_wait=True, lookahead=True):
            wait_gather(b)
            start_store(i, b)
            if lookahead:
                nb = (b + _LOOK) % _NBUF
                if store_wait:
                    wait_store(nb)
                start_gather(i + _LOOK, nb)

        for b in range(_LOOK):
            start_gather(b, b)
        for i in range(pro):
            step(i, i % _NBUF, store_wait=False)

        n_main = (n_chunks - _LOOK - pro) // _NBUF

        def body(t, carry):
            for j in range(_NBUF):
                i = pro + _NBUF * t + j
                step(i, (pro + j) % _NBUF)
            return carry

        lax.fori_loop(0, n_main, body, 0)

        for i in range(pro + _NBUF * n_main, n_chunks):
            step(i, i % _NBUF,
                 store_wait=(i + _LOOK >= _NBUF),
                 lookahead=(i + _LOOK < n_chunks))
        for i in range(n_chunks - _NBUF, n_chunks):
            wait_store(i % _NBUF)

    return gather_kernel


@jax.jit
def kernel(x, embedding):
    b, s = x.shape
    total = b * s
    flat = x.reshape(32, total // (32 * _C), _C)
    out = _make_gather(total)(flat, embedding)
    return out.reshape(b, s, _D)


# speedup vs baseline: 1.0274x; 1.0037x over previous
import functools

import jax
import jax.numpy as jnp
from jax import lax
from jax.experimental import pallas as pl
from jax.experimental.pallas import tpu as pltpu, tpu_sc as plsc

_D = 128
_C = 128
_NBUF = 7
_LOOK = 6


@functools.lru_cache(maxsize=None)
def _make_gather(total):
    info = plsc.get_sparse_core_info()
    nc, ns = info.num_cores, info.num_subcores
    nw = nc * ns
    b_per_w = total // nw
    n_chunks = b_per_w // _C
    pro = _NBUF - _LOOK
    assert n_chunks >= 2 * _NBUF
    mesh = plsc.VectorSubcoreMesh(core_axis_name="c", subcore_axis_name="s")

    @functools.partial(
        pl.kernel,
        mesh=mesh,
        out_type=jax.ShapeDtypeStruct((total, _D), jnp.float32),
        scratch_types=[
            pltpu.VMEM((n_chunks, _C), jnp.int32),
        ]
        + [pltpu.VMEM((_C, _D), jnp.float32) for _ in range(_NBUF)]
        + [pltpu.SemaphoreType.DMA for _ in range(2 * _NBUF)],
    )
    def gather_kernel(idx_hbm, table_hbm, out_hbm, idx_v, *rest):
        bufs = rest[:_NBUF]
        gs = rest[_NBUF:2 * _NBUF]
        ss = rest[2 * _NBUF:]
        wid = lax.axis_index("s") * nc + lax.axis_index("c")
        base = wid * b_per_w
        pltpu.sync_copy(idx_hbm.at[wid], idx_v)

        def start_gather(i, b):
            pltpu.async_copy(table_hbm.at[idx_v.at[i]], bufs[b], gs[b])

        def wait_gather(b):
            pltpu.make_async_copy(
                table_hbm.at[pl.ds(0, _C)], bufs[b], gs[b]).wait()

        def start_store(i, b):
            pltpu.async_copy(bufs[b], out_hbm.at[pl.ds(base + i * _C, _C)],
                             ss[b])

        def wait_store(b):
            pltpu.make_async_copy(bufs[b], out_hbm.at[pl.ds(base, _C)],
                                  ss[b]).wait()

        def step(i, b, store_wait=True, lookahead=True):
            wait_gather(b)
            start_store(i, b)
            if lookahead:
                nb = (b + _LOOK) % _NBUF
                if store_wait:
                    wait_store(nb)
                start_gather(i + _LOOK, nb)

        for b in range(_LOOK):
            start_gather(b, b)
        for i in range(pro):
            step(i, i % _NBUF, store_wait=False)

        n_main = (n_chunks - _LOOK - pro) // _NBUF

        def body(t, carry):
            for j in range(_NBUF):
                i = pro + _NBUF * t + j
                step(i, (pro + j) % _NBUF)
            return carry

        lax.fori_loop(0, n_main, body, 0)

        for i in range(pro + _NBUF * n_main, n_chunks):
            step(i, i % _NBUF,
                 store_wait=(i + _LOOK >= _NBUF),
                 lookahead=(i + _LOOK < n_chunks))
        for i in range(n_chunks - _NBUF, n_chunks):
            wait_store(i % _NBUF)

    return gather_kernel


@jax.jit
def kernel(x, embedding):
    b, s = x.shape
    total = b * s
    flat = x.reshape(32, total // (32 * _C), _C)
    out = _make_gather(total)(flat, embedding)
    return out.reshape(b, s, _D)
